# Initial kernel scaffold; baseline (speedup 1.0000x reference)
import jax
import jax.numpy as jnp
from jax.experimental import pallas as pl


def kernel(x, adj, W1, b1, W2, b2):
    raise NotImplementedError("write your pallas kernel here")


# speedup vs baseline: 1.0021x; 1.0021x over previous
import jax
import jax.numpy as jnp
from jax.experimental import pallas as pl
from jax.experimental.pallas import tpu as pltpu

_ROWS = 256


def _gcn1(adj_ref, x_ref, w1_ref, b1_ref, w2_ref, h2_ref):
    ax = jnp.dot(adj_ref[...], x_ref[...], preferred_element_type=jnp.float32)
    h = jax.lax.dot_general(ax, w1_ref[...], (((1,), (1,)), ((), ())),
                            preferred_element_type=jnp.float32)
    h = jnp.maximum(h + b1_ref[...], 0.0)
    h2_ref[...] = jax.lax.dot_general(h, w2_ref[...], (((1,), (1,)), ((), ())),
                                      preferred_element_type=jnp.float32)


def _gcn2(adj_ref, h2_ref, b2_ref, out_ref):
    logits = jnp.dot(adj_ref[...], h2_ref[...],
                     preferred_element_type=jnp.float32) + b2_ref[...]
    m = jnp.max(logits, axis=1, keepdims=True)
    s = logits - m
    lse = jnp.log(jnp.sum(jnp.exp(s), axis=1, keepdims=True))
    out_ref[...] = s - lse


def kernel(x, adj, W1, b1, W2, b2):
    n, in_f = x.shape
    hid = W1.shape[0]
    out_f = W2.shape[0]
    grid = (pl.cdiv(n, _ROWS),)
    b1r = b1.reshape(1, hid)
    b2r = b2.reshape(1, out_f)

    h2 = pl.pallas_call(
        _gcn1,
        grid=grid,
        in_specs=[
            pl.BlockSpec((_ROWS, n), lambda i: (i, 0)),
            pl.BlockSpec((n, in_f), lambda i: (0, 0)),
            pl.BlockSpec((hid, in_f), lambda i: (0, 0)),
            pl.BlockSpec((1, hid), lambda i: (0, 0)),
            pl.BlockSpec((out_f, hid), lambda i: (0, 0)),
        ],
        out_specs=pl.BlockSpec((_ROWS, out_f), lambda i: (i, 0)),
        out_shape=jax.ShapeDtypeStruct((n, out_f), jnp.float32),
        compiler_params=pltpu.CompilerParams(
            dimension_semantics=("parallel",)),
    )(adj, x, W1, b1r, W2)

    out = pl.pallas_call(
        _gcn2,
        grid=grid,
        in_specs=[
            pl.BlockSpec((_ROWS, n), lambda i: (i, 0)),
            pl.BlockSpec((n, out_f), lambda i: (0, 0)),
            pl.BlockSpec((1, out_f), lambda i: (0, 0)),
        ],
        out_specs=pl.BlockSpec((_ROWS, out_f), lambda i: (i, 0)),
        out_shape=jax.ShapeDtypeStruct((n, out_f), jnp.float32),
        compiler_params=pltpu.CompilerParams(
            dimension_semantics=("parallel",)),
    )(adj, h2, b2r)
    return out
